# Initial kernel scaffold; baseline (speedup 1.0000x reference)
import jax
import jax.numpy as jnp
from jax.experimental import pallas as pl


def kernel(x, edge_index, W1, b1, W2, b2, lin_W, lin_b):
    raise NotImplementedError("write your pallas kernel here")


# speedup vs baseline: 19.3581x; 19.3581x over previous
import functools

import jax
import jax.numpy as jnp
from jax import lax
from jax.experimental import pallas as pl
from jax.experimental.pallas import tpu as pltpu
from jax.experimental.pallas import tpu_sc as plsc

N = 10000
E = 320000
D = 128
N_PAD = 10240
NC = 2
NS = 16
NW = NC * NS
EPW = E // NW
CH = 80
NCH = EPW // CH
ZROWS = N_PAD // NS
L = 16
STR = N_PAD // NS

_mesh = plsc.VectorSubcoreMesh(
    core_axis_name="c", subcore_axis_name="s", num_cores=NC, num_subcores=NS)
_sc_params = pltpu.CompilerParams(needs_layout_passes=False)


def _zero_1d(ref, n):
    zeros = jnp.zeros((L,), jnp.float32)

    def body(j, _):
        ref[pl.ds(j * L, L)] = zeros
        return 0
    lax.fori_loop(0, n // L, body, 0)


def _reduce_tiles(local, stage, acc_buf, row_buf, out_row, sid):
    pltpu.sync_copy(local, stage.at[sid])
    plsc.subcore_barrier()
    pltpu.sync_copy(stage.at[0, pl.ds(sid * STR, STR)], acc_buf)
    for r in range(1, NS):
        pltpu.sync_copy(stage.at[r, pl.ds(sid * STR, STR)], row_buf)

        def body(j, _):
            acc_buf[pl.ds(j * L, L)] = (acc_buf[pl.ds(j * L, L)]
                                        + row_buf[pl.ds(j * L, L)])
            return 0
        lax.fori_loop(0, STR // L, body, 0)
    pltpu.sync_copy(acc_buf, out_row.at[pl.ds(sid * STR, STR)])


@functools.partial(
    pl.kernel,
    out_type=jax.ShapeDtypeStruct((NC, N_PAD), jnp.float32),
    mesh=_mesh,
    scratch_types=[
        pltpu.VMEM((N_PAD,), jnp.float32),
        pltpu.VMEM((CH,), jnp.int32),
        pltpu.VMEM((STR,), jnp.float32),
        pltpu.VMEM((STR,), jnp.float32),
        pltpu.VMEM_SHARED((NS, N_PAD), jnp.float32),
    ],
    compiler_params=_sc_params,
)
def _deg_kernel(dst_hbm, deg_out, deg_local, dst_buf, acc_buf, row_buf, stage):
    cid = lax.axis_index("c")
    sid = lax.axis_index("s")
    wid = sid * NC + cid
    ones = jnp.ones((L,), jnp.float32)

    _zero_1d(deg_local, N_PAD)

    def chunk_body(ci, _):
        base = wid * EPW + ci * CH
        pltpu.sync_copy(dst_hbm.at[pl.ds(base, CH)], dst_buf)

        def vec_body(j, _):
            idx = dst_buf[pl.ds(j * L, L)]
            plsc.addupdate_scatter(deg_local, [idx], ones)
            return 0
        lax.fori_loop(0, CH // L, vec_body, 0)
        return 0
    lax.fori_loop(0, NCH, chunk_body, 0)

    _reduce_tiles(deg_local, stage, acc_buf, row_buf, deg_out.at[cid], sid)


@functools.partial(
    pl.kernel,
    out_type=(
        jax.ShapeDtypeStruct((NC, N_PAD, D), jnp.float32),
        jax.ShapeDtypeStruct((NC, N_PAD), jnp.float32),
    ),
    mesh=_mesh,
    scratch_types=[
        pltpu.VMEM((N,), jnp.float32),
        pltpu.VMEM((N_PAD,), jnp.float32),
        pltpu.VMEM((CH,), jnp.int32),
        pltpu.VMEM((CH,), jnp.int32),
        pltpu.VMEM((CH, D), jnp.float32),
        pltpu.VMEM((STR,), jnp.float32),
        pltpu.VMEM((STR,), jnp.float32),
        pltpu.VMEM_SHARED((N_PAD, D), jnp.float32),
        pltpu.VMEM_SHARED((NS, N_PAD), jnp.float32),
        pltpu.SemaphoreType.DMA,
        pltpu.SemaphoreType.DMA,
    ],
    compiler_params=_sc_params,
)
def _edge_kernel(src_hbm, dst_hbm, dinv_hbm, g_hbm, z_out, s_out,
                 dinv_local, s_local, src_buf, dst_buf, rows,
                 acc_buf, row_buf, z_shared, stage, sem_g, sem_z):
    cid = lax.axis_index("c")
    sid = lax.axis_index("s")
    wid = sid * NC + cid
    zeros = jnp.zeros((L,), jnp.float32)

    pltpu.sync_copy(dinv_hbm, dinv_local)
    _zero_1d(s_local, N_PAD)

    def zero_rows(r, _):
        for k in range(D // L):
            rows[r, pl.ds(k * L, L)] = zeros
        return 0
    lax.fori_loop(0, CH, zero_rows, 0)

    for t in range(ZROWS // CH):
        pltpu.sync_copy(rows.at[pl.ds(0, CH)],
                        z_shared.at[pl.ds(sid * ZROWS + t * CH, CH)])
    plsc.subcore_barrier()

    def _run():
        def chunk_body(ci, _):
            base = wid * EPW + ci * CH
            pltpu.sync_copy(src_hbm.at[pl.ds(base, CH)], src_buf)
            pltpu.sync_copy(dst_hbm.at[pl.ds(base, CH)], dst_buf)
            pltpu.async_copy(g_hbm.at[src_buf], rows, sem_g).wait()

            def vec_body(j, _):
                sidx = src_buf[pl.ds(j * L, L)]
                didx = dst_buf[pl.ds(j * L, L)]
                dd = plsc.load_gather(dinv_local, [didx])
                plsc.addupdate_scatter(s_local, [sidx], dd)
                for r in range(L):
                    w = dd[r]
                    row = j * L + r
                    for k in range(D // L):
                        rows[row, pl.ds(k * L, L)] = (
                            rows[row, pl.ds(k * L, L)] * w)
                return 0
            lax.fori_loop(0, CH // L, vec_body, 0)

            pltpu.async_copy(rows, z_shared.at[dst_buf], sem_z,
                             add=True).wait()
            return 0
        lax.fori_loop(0, NCH, chunk_body, 0)

    _run()
    plsc.subcore_barrier()

    pltpu.sync_copy(z_shared.at[pl.ds(sid * ZROWS, ZROWS)],
                    z_out.at[cid, pl.ds(sid * ZROWS, ZROWS)])
    _reduce_tiles(s_local, stage, acc_buf, row_buf, s_out.at[cid], sid)


def _tc_a_body(x_blk, w1, d0, d1, g_out, dinv_out):
    deg = d0[...] + d1[...] + 1.0
    dinv = lax.rsqrt(deg)
    h0 = jnp.dot(x_blk[...], w1[...], preferred_element_type=jnp.float32)
    g_out[...] = dinv * h0
    dinv_out[...] = dinv


def _tc_a(x, W1, d0, d1):
    blk = 2000
    grid = N // blk
    return pl.pallas_call(
        _tc_a_body,
        grid=(grid,),
        in_specs=[
            pl.BlockSpec((blk, D), lambda i: (i, 0)),
            pl.BlockSpec((D, D), lambda i: (0, 0)),
            pl.BlockSpec((blk, 1), lambda i: (i, 0)),
            pl.BlockSpec((blk, 1), lambda i: (i, 0)),
        ],
        out_specs=[
            pl.BlockSpec((blk, D), lambda i: (i, 0)),
            pl.BlockSpec((blk, 1), lambda i: (i, 0)),
        ],
        out_shape=[
            jax.ShapeDtypeStruct((N, D), jnp.float32),
            jax.ShapeDtypeStruct((N, 1), jnp.float32),
        ],
    )(x, W1, d0, d1)


def _tc_b_body(z0, z1, g, dinv, s0, s1, b1, w2, b2, linw, linb, out, acc):
    i = pl.program_id(0)

    @pl.when(i == 0)
    def _():
        acc[...] = jnp.zeros_like(acc)

    dv = dinv[...]
    h1 = jnp.maximum(z0[...] + z1[...] + dv * g[...] + b1[...], 0.0)
    c = dv * (dv + s0[...] + s1[...])
    acc[...] += jnp.sum(c * h1, axis=0, keepdims=True)

    @pl.when(i == pl.num_programs(0) - 1)
    def _():
        pooled = jnp.dot(acc[...] * (1.0 / N), w2[...],
                         preferred_element_type=jnp.float32) + b2[...]
        out[...] = jnp.dot(pooled, linw[...],
                           preferred_element_type=jnp.float32) + linb[...]


def _tc_b(z0, z1, g, dinv, s0, s1, b1, W2, b2, linw_pad, linb_pad):
    blk = 2000
    grid = N // blk
    full = lambda i: (0, 0)
    row = lambda i: (i, 0)
    return pl.pallas_call(
        _tc_b_body,
        grid=(grid,),
        in_specs=[
            pl.BlockSpec((blk, D), row),
            pl.BlockSpec((blk, D), row),
            pl.BlockSpec((blk, D), row),
            pl.BlockSpec((blk, 1), row),
            pl.BlockSpec((blk, 1), row),
            pl.BlockSpec((blk, 1), row),
            pl.BlockSpec((1, D), full),
            pl.BlockSpec((D, D), full),
            pl.BlockSpec((1, D), full),
            pl.BlockSpec((D, D), full),
            pl.BlockSpec((1, D), full),
        ],
        out_specs=pl.BlockSpec((1, D), full),
        out_shape=jax.ShapeDtypeStruct((1, D), jnp.float32),
        scratch_shapes=[pltpu.VMEM((1, D), jnp.float32)],
    )(z0, z1, g, dinv, s0, s1, b1, W2, b2, linw_pad, linb_pad)


def kernel(x, edge_index, W1, b1, W2, b2, lin_W, lin_b):
    src = edge_index[0]
    dst = edge_index[1]

    deg_p = _deg_kernel(dst)
    d0 = deg_p[0, :N].reshape(N, 1)
    d1 = deg_p[1, :N].reshape(N, 1)

    g, dinv2d = _tc_a(x, W1, d0, d1)
    dinv = dinv2d.reshape(N)

    z_p, s_p = _edge_kernel(src, dst, dinv, g)
    s0 = s_p[0, :N].reshape(N, 1)
    s1 = s_p[1, :N].reshape(N, 1)

    linw_pad = jnp.zeros((D, D), jnp.float32).at[:, :2].set(lin_W)
    linb_pad = jnp.zeros((1, D), jnp.float32).at[0, :2].set(lin_b)

    out = _tc_b(z_p[0, :N], z_p[1, :N], g, dinv2d, s0, s1, b1.reshape(1, D),
                W2, b2.reshape(1, D), linw_pad, linb_pad)
    return out[:, :2]
